# baseline (speedup 1.0000x reference)
import functools

import jax
import jax.numpy as jnp
from jax import lax
from jax.experimental import pallas as pl
from jax.experimental.pallas import tpu as pltpu
from jax.experimental.pallas import tpu_sc as plsc

R = 59
B = 1024
K = 10
D = 128
OUT = 128

NC = 2
NS = 16
NW = NC * NS

SEGS = R * B
HALF = 32
CH0 = 2
CH1 = 58
NH_TOTAL = 2 * NS * (CH0 + CH1)
H0_TOTAL = 2 * NS * CH0
SEGS_PAD = NH_TOTAL * HALF
GH = 4
GS = (HALF * K) // GH


def _sc_gather_sum(idx_arr, features):
    mesh = plsc.VectorSubcoreMesh(core_axis_name="c", subcore_axis_name="s")

    @functools.partial(
        pl.kernel,
        out_type=jax.ShapeDtypeStruct((SEGS_PAD, D), jnp.float32),
        mesh=mesh,
        scratch_types=[
            pltpu.VMEM((GH, GS), jnp.int32),
            pltpu.VMEM((GH, GS), jnp.int32),
            pltpu.VMEM((HALF * K, D), jnp.float32),
            pltpu.VMEM((HALF * K, D), jnp.float32),
            pltpu.VMEM((HALF, D), jnp.float32),
            pltpu.VMEM((HALF, D), jnp.float32),
            pltpu.SemaphoreType.DMA,
            pltpu.SemaphoreType.DMA,
            pltpu.SemaphoreType.DMA,
            pltpu.SemaphoreType.DMA,
        ],
    )
    def body(idx_hbm, feat_hbm, out_hbm, idx_a, idx_b, rows_a, rows_b,
             sums_a, sums_b, sem_a, sem_b, osem_a, osem_b):
        cid = lax.axis_index("c")
        sid = lax.axis_index("s")
        h0 = jnp.where(cid == 0, sid * (2 * CH0), H0_TOTAL + sid * (2 * CH1))
        nch = jnp.where(cid == 0, CH0, CH1)

        def fire(h, idx_v, rows_v, sem):
            pltpu.sync_copy(idx_hbm.at[h0 + h], idx_v)
            for g in range(GH):
                pltpu.async_copy(
                    feat_hbm.at[idx_v.at[g]],
                    rows_v.at[pl.ds(g * GS, GS)],
                    sem,
                )

        def drain(idx_v, rows_v, sem):
            with jax.named_scope("sc_drain"):
                for g in range(GH):
                    pltpu.make_async_copy(
                        feat_hbm.at[idx_v.at[g]],
                        rows_v.at[pl.ds(g * GS, GS)],
                        sem,
                    ).wait()

        def accumulate(rows_v, sums_v):
            with jax.named_scope("sc_accum"):
                _accum_loop(rows_v, sums_v)

        def _accum_loop(rows_v, sums_v):
            @plsc.parallel_loop(0, HALF, step=1, unroll=4)
            def seg_body(s):
                base = s * K
                acc = [rows_v[base, pl.ds(j * 16, 16)] for j in range(D // 16)]
                for kk in range(1, K):
                    for j in range(D // 16):
                        acc[j] = acc[j] + rows_v[base + kk, pl.ds(j * 16, 16)]
                for j in range(D // 16):
                    sums_v[s, pl.ds(j * 16, 16)] = acc[j]

        def out_slot(h):
            return out_hbm.at[pl.ds((h0 + h) * HALF, HALF)]

        fire(0, idx_a, rows_a, sem_a)

        def chunk_body(c, carry):
            ha = 2 * c
            hb = 2 * c + 1

            fire(hb, idx_b, rows_b, sem_b)
            drain(idx_a, rows_a, sem_a)

            @pl.when(c > 0)
            def _():
                pltpu.make_async_copy(sums_a, out_slot(ha - 2), osem_a).wait()

            accumulate(rows_a, sums_a)
            pltpu.async_copy(sums_a, out_slot(ha), osem_a)

            @pl.when(c + 1 < nch)
            def _():
                fire(ha + 2, idx_a, rows_a, sem_a)

            drain(idx_b, rows_b, sem_b)

            @pl.when(c > 0)
            def _():
                pltpu.make_async_copy(sums_b, out_slot(hb - 2), osem_b).wait()

            accumulate(rows_b, sums_b)
            pltpu.async_copy(sums_b, out_slot(hb), osem_b)
            return carry

        lax.fori_loop(0, nch, chunk_body, 0)
        pltpu.make_async_copy(sums_a, out_slot(2 * nch - 2), osem_a).wait()
        pltpu.make_async_copy(sums_b, out_slot(2 * nch - 1), osem_b).wait()

    return body(idx_arr, features)


def _tc_body(s_ref, wagg_ref, bagg_ref, whet_ref, bhet_ref, out_ref):
    r = pl.program_id(0)
    z = jnp.dot(s_ref[...], wagg_ref[0], preferred_element_type=jnp.float32)
    z = z * (1.0 / K) + bagg_ref[0]
    act = jnp.where(z > 0, z, 0.01 * z)
    contrib = jnp.dot(act, whet_ref[0], preferred_element_type=jnp.float32)

    @pl.when(r == 0)
    def _():
        out_ref[...] = contrib + bhet_ref[...]

    @pl.when(r > 0)
    def _():
        out_ref[...] += contrib

    @pl.when(r == R - 1)
    def _():
        out_ref[...] = jax.nn.sigmoid(out_ref[...])


def _tc_finish(sums, W_agg, b_agg, W_het, b_het):
    return pl.pallas_call(
        _tc_body,
        grid=(R,),
        in_specs=[
            pl.BlockSpec((B, D), lambda r: (r, 0)),
            pl.BlockSpec((1, D, OUT), lambda r: (r, 0, 0)),
            pl.BlockSpec((1, 1, OUT), lambda r: (r, 0, 0)),
            pl.BlockSpec((1, OUT, OUT), lambda r: (r, 0, 0)),
            pl.BlockSpec((1, OUT), lambda r: (0, 0)),
        ],
        out_specs=pl.BlockSpec((B, OUT), lambda r: (0, 0)),
        out_shape=jax.ShapeDtypeStruct((B, OUT), jnp.float32),
    )(sums, W_agg, b_agg, W_het, b_het)


def kernel(gid_batch, neigh_idx, features, W_agg, b_agg, W_het, b_het):
    flat = neigh_idx.reshape(SEGS, K).astype(jnp.int32)
    if SEGS_PAD >= SEGS:
        pad = jnp.zeros((SEGS_PAD - SEGS, K), jnp.int32)
        flat = jnp.concatenate([flat, pad], axis=0)
    else:
        flat = flat[:SEGS_PAD]
    idx_arr = flat.reshape(NH_TOTAL, GH, GS)
    sums = _sc_gather_sum(idx_arr, features)
    return _tc_finish(
        sums,
        W_agg,
        b_agg.reshape(R, 1, OUT),
        W_het.reshape(R, OUT, OUT),
        b_het.reshape(1, OUT),
    )


# speedup vs baseline: 3.1431x; 3.1431x over previous
import functools

import jax
import jax.numpy as jnp
from jax import lax
from jax.experimental import pallas as pl
from jax.experimental.pallas import tpu as pltpu
from jax.experimental.pallas import tpu_sc as plsc

R = 59
B = 1024
K = 10
D = 128
OUT = 128

NC = 2
NS = 16
NW = NC * NS

SEGS = R * B
HALF = 32
CH0 = 30
CH1 = 30
NH_TOTAL = 2 * NS * (CH0 + CH1)
H0_TOTAL = 2 * NS * CH0
SEGS_PAD = NH_TOTAL * HALF
GH = 4
GS = (HALF * K) // GH


def _sc_gather_sum(idx_arr, features):
    mesh = plsc.VectorSubcoreMesh(core_axis_name="c", subcore_axis_name="s")

    @functools.partial(
        pl.kernel,
        out_type=jax.ShapeDtypeStruct((SEGS_PAD, D), jnp.float32),
        mesh=mesh,
        scratch_types=[
            pltpu.VMEM((GH, GS), jnp.int32),
            pltpu.VMEM((GH, GS), jnp.int32),
            pltpu.VMEM((HALF * K, D), jnp.float32),
            pltpu.VMEM((HALF * K, D), jnp.float32),
            pltpu.VMEM((HALF, D), jnp.float32),
            pltpu.VMEM((HALF, D), jnp.float32),
            pltpu.SemaphoreType.DMA,
            pltpu.SemaphoreType.DMA,
            pltpu.SemaphoreType.DMA,
            pltpu.SemaphoreType.DMA,
        ],
    )
    def body(idx_hbm, feat_hbm, out_hbm, idx_a, idx_b, rows_a, rows_b,
             sums_a, sums_b, sem_a, sem_b, osem_a, osem_b):
        cid = lax.axis_index("c")
        sid = lax.axis_index("s")
        h0 = jnp.where(cid == 0, sid * (2 * CH0), H0_TOTAL + sid * (2 * CH1))
        nch = jnp.where(cid == 0, CH0, CH1)

        def fire(h, idx_v, rows_v, sem):
            pltpu.sync_copy(idx_hbm.at[h0 + h], idx_v)
            for g in range(GH):
                pltpu.async_copy(
                    feat_hbm.at[idx_v.at[g]],
                    rows_v.at[pl.ds(g * GS, GS)],
                    sem,
                )

        def drain(idx_v, rows_v, sem):
            with jax.named_scope("sc_drain"):
                for g in range(GH):
                    pltpu.make_async_copy(
                        feat_hbm.at[idx_v.at[g]],
                        rows_v.at[pl.ds(g * GS, GS)],
                        sem,
                    ).wait()

        def accumulate(rows_v, sums_v):
            with jax.named_scope("sc_accum"):
                _accum_loop(rows_v, sums_v)

        def _accum_loop(rows_v, sums_v):
            @plsc.parallel_loop(0, HALF, step=1, unroll=4)
            def seg_body(s):
                base = s * K
                acc = [rows_v[base, pl.ds(j * 16, 16)] for j in range(D // 16)]
                for kk in range(1, K):
                    for j in range(D // 16):
                        acc[j] = acc[j] + rows_v[base + kk, pl.ds(j * 16, 16)]
                for j in range(D // 16):
                    sums_v[s, pl.ds(j * 16, 16)] = acc[j]

        def out_slot(h):
            return out_hbm.at[pl.ds((h0 + h) * HALF, HALF)]

        fire(0, idx_a, rows_a, sem_a)

        def chunk_body(c, carry):
            ha = 2 * c
            hb = 2 * c + 1

            fire(hb, idx_b, rows_b, sem_b)
            drain(idx_a, rows_a, sem_a)

            @pl.when(c > 0)
            def _():
                pltpu.make_async_copy(sums_a, out_slot(ha - 2), osem_a).wait()

            accumulate(rows_a, sums_a)
            pltpu.async_copy(sums_a, out_slot(ha), osem_a)

            @pl.when(c + 1 < nch)
            def _():
                fire(ha + 2, idx_a, rows_a, sem_a)

            drain(idx_b, rows_b, sem_b)

            @pl.when(c > 0)
            def _():
                pltpu.make_async_copy(sums_b, out_slot(hb - 2), osem_b).wait()

            accumulate(rows_b, sums_b)
            pltpu.async_copy(sums_b, out_slot(hb), osem_b)
            return carry

        lax.fori_loop(0, nch, chunk_body, 0)
        pltpu.make_async_copy(sums_a, out_slot(2 * nch - 2), osem_a).wait()
        pltpu.make_async_copy(sums_b, out_slot(2 * nch - 1), osem_b).wait()

    return body(idx_arr, features)


def _tc_body(s_ref, wagg_ref, bagg_ref, whet_ref, bhet_ref, out_ref):
    r = pl.program_id(0)
    z = jnp.dot(s_ref[...], wagg_ref[0], preferred_element_type=jnp.float32)
    z = z * (1.0 / K) + bagg_ref[0]
    act = jnp.where(z > 0, z, 0.01 * z)
    contrib = jnp.dot(act, whet_ref[0], preferred_element_type=jnp.float32)

    @pl.when(r == 0)
    def _():
        out_ref[...] = contrib + bhet_ref[...]

    @pl.when(r > 0)
    def _():
        out_ref[...] += contrib

    @pl.when(r == R - 1)
    def _():
        out_ref[...] = jax.nn.sigmoid(out_ref[...])


def _tc_finish(sums, W_agg, b_agg, W_het, b_het):
    return pl.pallas_call(
        _tc_body,
        grid=(R,),
        in_specs=[
            pl.BlockSpec((B, D), lambda r: (r, 0)),
            pl.BlockSpec((1, D, OUT), lambda r: (r, 0, 0)),
            pl.BlockSpec((1, 1, OUT), lambda r: (r, 0, 0)),
            pl.BlockSpec((1, OUT, OUT), lambda r: (r, 0, 0)),
            pl.BlockSpec((1, OUT), lambda r: (0, 0)),
        ],
        out_specs=pl.BlockSpec((B, OUT), lambda r: (0, 0)),
        out_shape=jax.ShapeDtypeStruct((B, OUT), jnp.float32),
    )(sums, W_agg, b_agg, W_het, b_het)


def kernel(gid_batch, neigh_idx, features, W_agg, b_agg, W_het, b_het):
    flat = neigh_idx.reshape(SEGS, K).astype(jnp.int32)
    npad = SEGS_PAD - SEGS
    pad = (jnp.arange(npad * K, dtype=jnp.int32) % features.shape[0]).reshape(
        npad, K)
    idx_arr = jnp.concatenate([flat, pad], axis=0).reshape(NH_TOTAL, GH, GS)
    sums = _sc_gather_sum(idx_arr, features)
    return _tc_finish(
        sums,
        W_agg,
        b_agg.reshape(R, 1, OUT),
        W_het.reshape(R, OUT, OUT),
        b_het.reshape(1, OUT),
    )
